# baseline (speedup 1.0000x reference)
import functools

import jax
import jax.numpy as jnp
from jax import lax
from jax.experimental import pallas as pl
from jax.experimental.pallas import tpu as pltpu
from jax.experimental.pallas import tpu_sc as plsc

EMB = 64
BATCH = 16384
NC = 2
NS = 16
NW = NC * NS
BPW = BATCH // NW
IDXW = 128
CH = BPW // IDXW

_sc_mesh = plsc.VectorSubcoreMesh(core_axis_name="c", subcore_axis_name="s")


@functools.partial(
    pl.kernel,
    out_type=[
        jax.ShapeDtypeStruct((BATCH, EMB), jnp.float32),
        jax.ShapeDtypeStruct((BATCH, EMB), jnp.float32),
    ],
    mesh=_sc_mesh,
    scratch_types=[
        pltpu.VMEM((BPW,), jnp.int32),
        pltpu.VMEM((BPW, EMB), jnp.float32),
        pltpu.SemaphoreType.DMA,
    ],
)
def _sc_gather(uidx_hbm, iidx_hbm, utab_hbm, itab_hbm, urows_hbm, irows_hbm,
               idx_v, rows_v, sem):
    wid = lax.axis_index("s") * NC + lax.axis_index("c")
    base = wid * BPW
    L = 16

    def one_table(idx_hbm, tab_hbm, out_hbm):
        pltpu.sync_copy(idx_hbm.at[pl.ds(base, BPW)], idx_v)

        def chunk(k, carry):
            vec = idx_v[pl.ds(k * L, L)]
            for l in range(L):
                pltpu.async_copy(tab_hbm.at[vec[l]], rows_v.at[k * L + l], sem)
            return carry

        lax.fori_loop(0, BPW // L, chunk, 0)
        pltpu.make_async_copy(out_hbm.at[pl.ds(base, BPW)], rows_v, sem).wait()
        pltpu.sync_copy(rows_v, out_hbm.at[pl.ds(base, BPW)])

    one_table(uidx_hbm, utab_hbm, urows_hbm)
    one_table(iidx_hbm, itab_hbm, irows_hbm)


ROWS = 1024
GRID = BATCH // ROWS


def _mlp_body(u_ref, i_ref, w1a_ref, w1b_ref, b1_ref, w2_ref, b2_ref,
              w3_ref, b3_ref, out_ref):
    h = jnp.dot(u_ref[...], w1a_ref[...], preferred_element_type=jnp.float32)
    h = h + jnp.dot(i_ref[...], w1b_ref[...], preferred_element_type=jnp.float32)
    h = jnp.maximum(h + b1_ref[...], 0.0)
    h = jnp.maximum(
        jnp.dot(h, w2_ref[...], preferred_element_type=jnp.float32) + b2_ref[...],
        0.0)
    out_ref[...] = (
        jnp.dot(h, w3_ref[...], preferred_element_type=jnp.float32) + b3_ref[...])


_tc_mlp = pl.pallas_call(
    _mlp_body,
    grid=(GRID,),
    in_specs=[
        pl.BlockSpec((ROWS, EMB), lambda i: (i, 0)),
        pl.BlockSpec((ROWS, EMB), lambda i: (i, 0)),
        pl.BlockSpec((EMB, 64), lambda i: (0, 0)),
        pl.BlockSpec((EMB, 64), lambda i: (0, 0)),
        pl.BlockSpec((1, 64), lambda i: (0, 0)),
        pl.BlockSpec((64, 32), lambda i: (0, 0)),
        pl.BlockSpec((1, 32), lambda i: (0, 0)),
        pl.BlockSpec((32, 1), lambda i: (0, 0)),
        pl.BlockSpec((1, 1), lambda i: (0, 0)),
    ],
    out_specs=pl.BlockSpec((ROWS, 1), lambda i: (i, 0)),
    out_shape=jax.ShapeDtypeStruct((BATCH, 1), jnp.float32),
)


def kernel(user, item, user_table, item_table, W1, b1, W2, b2, W3, b3):
    uidx = user.astype(jnp.int32)
    iidx = item.astype(jnp.int32)
    urows, irows = _sc_gather(uidx, iidx, user_table, item_table)
    out = _tc_mlp(urows, irows, W1[:EMB], W1[EMB:], b1.reshape(1, EMB),
                  W2, b2.reshape(1, 32), W3, b3.reshape(1, 1))
    return out.reshape(BATCH)
